# baseline (speedup 1.0000x reference)
import functools

import jax
import jax.numpy as jnp
from jax import lax
from jax.experimental import pallas as pl
from jax.experimental.pallas import tpu as pltpu
from jax.experimental.pallas import tpu_sc as plsc

_NC = 2
_NS = 16
_NW = _NC * _NS
_BN = 512
_CH = 48
_NB = 3
_SCK = 6


def _table_body(h_ref, w_ref, g_ref, o_ref):
    hb = h_ref[...]
    w = w_ref[0]
    g = g_ref[0]
    t = jnp.dot(hb, w, preferred_element_type=jnp.float32)
    logit = jnp.sum(hb * g, axis=1, keepdims=True)
    o_ref[...] = t * jax.nn.sigmoid(logit)


def _make_table(hpad, W, gate3d, npad, d, r):
    bnt = 2048
    nb_per_r = npad // bnt
    return pl.pallas_call(
        _table_body,
        grid=(nb_per_r, r),
        in_specs=[
            pl.BlockSpec((bnt, d), lambda nb, ri: (nb, 0)),
            pl.BlockSpec((1, d, d), lambda nb, ri: (ri, 0, 0)),
            pl.BlockSpec((1, 1, d), lambda nb, ri: (ri, 0, 0)),
        ],
        out_specs=pl.BlockSpec(
            (bnt, d), lambda nb, ri: (ri * nb_per_r + nb, 0)),
        out_shape=jax.ShapeDtypeStruct((r * npad, d), jnp.float32),
    )(hpad, W, gate3d)


def _edge_prep_body(src_ref, rel_ref, gidx_ref, *, npad):
    gidx_ref[...] = rel_ref[...] * npad + src_ref[...]


def _edge_prep(src, rel, epad, npad):
    er = epad // 128
    return pl.pallas_call(
        functools.partial(_edge_prep_body, npad=npad),
        out_shape=jax.ShapeDtypeStruct((er, 128), jnp.int32),
    )(src.reshape(er, 128), rel.reshape(er, 128))


def _combine_body(p_ref, h_ref, loop_ref, b_ref, o_ref, *, act):
    acc = p_ref[0] + p_ref[1] + b_ref[...] + jnp.dot(
        h_ref[...], loop_ref[...], preferred_element_type=jnp.float32)
    if act:
        acc = jnp.maximum(acc, 0.0)
    o_ref[...] = acc


def _combine(partials, hpad, loopW, b2d, act, npad, d):
    return pl.pallas_call(
        functools.partial(_combine_body, act=act),
        grid=(npad // _BN,),
        in_specs=[
            pl.BlockSpec((2, _BN, d), lambda nb: (0, nb, 0)),
            pl.BlockSpec((_BN, d), lambda nb: (nb, 0)),
            pl.BlockSpec((d, d), lambda nb: (0, 0)),
            pl.BlockSpec((1, d), lambda nb: (0, 0)),
        ],
        out_specs=pl.BlockSpec((_BN, d), lambda nb: (nb, 0)),
        out_shape=jax.ShapeDtypeStruct((npad, d), jnp.float32),
    )(partials, hpad, loopW, b2d)


def _edge_agg(gidx, dst, norm, table_flat, npad, d):
    epad = gidx.shape[0]
    epw = epad // _NW
    nch = epw // _CH
    rpt = npad // _NS
    mesh = plsc.VectorSubcoreMesh(
        core_axis_name="c", subcore_axis_name="s", num_cores=_NC)

    sup = _SCK * _CH
    nsup = nch // _SCK
    cpb = 2 * _SCK

    scratch = (
        [pltpu.VMEM((sup,), jnp.int32)] * 2 +
        [pltpu.VMEM((sup,), jnp.int32)] * 2 +
        [pltpu.VMEM((sup,), jnp.float32)] * 2 +
        [pltpu.VMEM((_CH,), jnp.int32)] * _NB +
        [pltpu.VMEM((_CH, d), jnp.float32)] * _NB +
        [pltpu.VMEM((8, d), jnp.float32)] +
        [pltpu.VMEM_SHARED((npad, d), jnp.float32)] +
        [pltpu.SemaphoreType.DMA] * (2 + 2 * _NB)
    )

    @functools.partial(
        pl.kernel, mesh=mesh,
        out_type=jax.ShapeDtypeStruct((_NC, npad, d), jnp.float32),
        scratch_types=scratch,
    )
    def body(gidx_h, dst_h, norm_h, tab_h, out_h, *scr):
        cid = lax.axis_index("c")
        sid = lax.axis_index("s")
        wid = sid * _NC + cid
        wbase = wid * epw
        gsup = scr[0:2]
        dsup = scr[2:4]
        nsb = scr[4:6]
        dcs = scr[6:6 + _NB]
        rows = scr[6 + _NB:6 + 2 * _NB]
        zero_v = scr[6 + 2 * _NB]
        agg_s = scr[7 + 2 * _NB]
        semm = scr[8 + 2 * _NB:10 + 2 * _NB]
        semr = scr[10 + 2 * _NB:10 + 3 * _NB]
        sems = scr[10 + 3 * _NB:10 + 4 * _NB]

        zvec = jnp.zeros((16,), jnp.float32)
        for i in range(8):
            for j in range(d // 16):
                zero_v[i, pl.ds(j * 16, 16)] = zvec

        def zloop(i, c):
            pltpu.sync_copy(zero_v, agg_s.at[pl.ds(sid * rpt + i * 8, 8)])
            return c
        lax.fori_loop(0, rpt // 8, zloop, 0)
        plsc.subcore_barrier()

        def meta_issue(s, mb):
            base = pl.ds(wbase + s * sup, sup)
            pltpu.async_copy(gidx_h.at[base], gsup[mb], semm[mb])
            pltpu.async_copy(dst_h.at[base], dsup[mb], semm[mb])
            pltpu.async_copy(norm_h.at[base], nsb[mb], semm[mb])

        def meta_wait(s, mb):
            base = pl.ds(wbase + s * sup, sup)
            pltpu.make_async_copy(gidx_h.at[base], gsup[mb], semm[mb]).wait()
            pltpu.make_async_copy(dst_h.at[base], dsup[mb], semm[mb]).wait()
            pltpu.make_async_copy(norm_h.at[base], nsb[mb], semm[mb]).wait()

        def rows_issue(rb, mb, k):
            idx = gsup[mb].at[pl.ds(k * _CH, _CH)]
            pltpu.async_copy(tab_h.at[idx], rows[rb], semr[rb])

        def rows_wait(rb, mb, k):
            idx = gsup[mb].at[pl.ds(k * _CH, _CH)]
            pltpu.make_async_copy(tab_h.at[idx], rows[rb], semr[rb]).wait()

        def scat_issue(rb, mb, k):
            for t in range(_CH // 16):
                dcs[rb][pl.ds(t * 16, 16)] = (
                    dsup[mb][pl.ds(k * _CH + t * 16, 16)])
            pltpu.async_copy(rows[rb], agg_s.at[dcs[rb]], sems[rb], add=True)

        def scat_wait(rb):
            pltpu.make_async_copy(rows[rb], agg_s.at[dcs[rb]],
                                  sems[rb]).wait()

        dnums = lax.GatherDimensionNumbers(
            offset_dims=(), collapsed_slice_dims=(0,),
            start_index_map=(0,))

        def scale(rb, mb, k):
            def sbody(g, c2):
                nvec = nsb[mb][pl.ds(k * _CH + g * 16, 16)]
                for l in range(16):
                    nv = lax.gather(
                        nvec, jnp.full((16, 1), l, jnp.int32), dnums,
                        slice_sizes=(1,),
                        mode=lax.GatherScatterMode.PROMISE_IN_BOUNDS)
                    e = g * 16 + l
                    for j in range(d // 16):
                        sl = pl.ds(j * 16, 16)
                        rows[rb][e, sl] = rows[rb][e, sl] * nv
                return c2
            lax.fori_loop(0, _CH // 16, sbody, 0)

        def step(i, j):
            cj = i * cpb + j
            rb = j % _NB
            mb = (j // _SCK) % 2
            k = j % _SCK
            nxtrb = (j + 1) % _NB

            @pl.when(cj >= _NB - 1)
            def _():
                scat_wait(nxtrb)

            if j == _SCK - 1:
                @pl.when(2 * i + 1 < nsup)
                def _():
                    meta_wait(2 * i + 1, 1)
            if j == cpb - 1:
                @pl.when(2 * i + 2 < nsup)
                def _():
                    meta_wait(2 * i + 2, 0)

            jn = j + 1
            nmb = (jn // _SCK) % 2 if jn < cpb else 0
            nk = jn % _SCK

            @pl.when(cj + 1 < nch)
            def _():
                rows_issue(nxtrb, nmb, nk)
            rows_wait(rb, mb, k)
            scale(rb, mb, k)
            scat_issue(rb, mb, k)

            if j == _SCK - 1:
                @pl.when(2 * i + 2 < nsup)
                def _():
                    meta_issue(2 * i + 2, 0)
            if j == cpb - 1:
                @pl.when(2 * i + 3 < nsup)
                def _():
                    meta_issue(2 * i + 3, 1)

        meta_issue(0, 0)
        meta_wait(0, 0)
        meta_issue(1, 1)
        rows_issue(0, 0, 0)

        def group(i, c):
            for j in range(cpb):
                step(i, j)
            return c
        lax.fori_loop(0, nsup // 2, group, 0)
        for rb in ((nch - 2) % _NB, (nch - 1) % _NB):
            scat_wait(rb)
        plsc.subcore_barrier()

        pltpu.sync_copy(agg_s.at[pl.ds(sid * rpt, rpt)],
                        out_h.at[cid, pl.ds(sid * rpt, rpt)])

    return body(gidx, dst, norm, table_flat)


def kernel(x, edge_index, rel_type, edge_norm, W0, b0, loop0, gate0,
           W1, b1, loop1, gate1):
    n, d = x.shape
    r = W0.shape[0]
    e = rel_type.shape[0]
    npad = -(-n // _BN) * _BN
    eunit = _NW * _CH * 2 * _SCK
    epad = -(-e // eunit) * eunit

    src = edge_index[0]
    dst = edge_index[1]
    rel = rel_type
    norm = edge_norm[:, 0]
    if epad != e:
        pad = epad - e
        src = jnp.pad(src, (0, pad))
        dst = jnp.pad(dst, (0, pad))
        rel = jnp.pad(rel, (0, pad))
        norm = jnp.pad(norm, (0, pad))
    hpad = jnp.pad(x, ((0, npad - n), (0, 0)))
    gidx = _edge_prep(src, rel, epad, npad).reshape(epad)

    t0 = _make_table(hpad, W0, gate0.reshape(r, 1, d), npad, d, r)
    p0 = _edge_agg(gidx, dst, norm, t0, npad, d)
    h1 = _combine(p0, hpad, loop0, b0.reshape(1, d), True, npad, d)

    t1 = _make_table(h1, W1, gate1.reshape(r, 1, d), npad, d, r)
    p1 = _edge_agg(gidx, dst, norm, t1, npad, d)
    out = _combine(p1, h1, loop1, b1.reshape(1, d), False, npad, d)
    return out[:n]


# speedup vs baseline: 2.1726x; 2.1726x over previous
import functools

import jax
import jax.numpy as jnp
from jax import lax
from jax.experimental import pallas as pl
from jax.experimental.pallas import tpu as pltpu
from jax.experimental.pallas import tpu_sc as plsc

_NC = 2
_NS = 16
_NW = _NC * _NS
_BN = 512
_CH = 48
_NB = 3


def _table_body(h_ref, w_ref, g_ref, o_ref):
    hb = h_ref[...]
    w = w_ref[0]
    g = g_ref[0]
    t = jnp.dot(hb, w, preferred_element_type=jnp.float32)
    logit = jnp.sum(hb * g, axis=1, keepdims=True)
    o_ref[...] = t * jax.nn.sigmoid(logit)


def _make_table(hpad, W, gate3d, npad, d, r):
    bnt = 2048
    nb_per_r = npad // bnt
    return pl.pallas_call(
        _table_body,
        grid=(nb_per_r, r),
        in_specs=[
            pl.BlockSpec((bnt, d), lambda nb, ri: (nb, 0)),
            pl.BlockSpec((1, d, d), lambda nb, ri: (ri, 0, 0)),
            pl.BlockSpec((1, 1, d), lambda nb, ri: (ri, 0, 0)),
        ],
        out_specs=pl.BlockSpec(
            (bnt, d), lambda nb, ri: (ri * nb_per_r + nb, 0)),
        out_shape=jax.ShapeDtypeStruct((r * npad, d), jnp.float32),
    )(hpad, W, gate3d)


def _edge_prep_body(src_ref, rel_ref, gidx_ref, *, npad):
    gidx_ref[...] = rel_ref[...] * npad + src_ref[...]


def _edge_prep(src, rel, epad, npad):
    er = epad // 128
    return pl.pallas_call(
        functools.partial(_edge_prep_body, npad=npad),
        out_shape=jax.ShapeDtypeStruct((er, 128), jnp.int32),
    )(src.reshape(er, 128), rel.reshape(er, 128))


def _combine_body(p_ref, h_ref, loop_ref, b_ref, o_ref, *, act):
    acc = p_ref[0] + p_ref[1] + b_ref[...] + jnp.dot(
        h_ref[...], loop_ref[...], preferred_element_type=jnp.float32)
    if act:
        acc = jnp.maximum(acc, 0.0)
    o_ref[...] = acc


def _combine(partials, hpad, loopW, b2d, act, npad, d):
    return pl.pallas_call(
        functools.partial(_combine_body, act=act),
        grid=(npad // _BN,),
        in_specs=[
            pl.BlockSpec((2, _BN, d), lambda nb: (0, nb, 0)),
            pl.BlockSpec((_BN, d), lambda nb: (nb, 0)),
            pl.BlockSpec((d, d), lambda nb: (0, 0)),
            pl.BlockSpec((1, d), lambda nb: (0, 0)),
        ],
        out_specs=pl.BlockSpec((_BN, d), lambda nb: (nb, 0)),
        out_shape=jax.ShapeDtypeStruct((npad, d), jnp.float32),
    )(partials, hpad, loopW, b2d)


def _edge_agg(gidx, dst, norm, table_flat, npad, d):
    epad = gidx.shape[0]
    epw = epad // _NW
    nch = epw // _CH
    rpt = npad // _NS
    mesh = plsc.VectorSubcoreMesh(
        core_axis_name="c", subcore_axis_name="s", num_cores=_NC)

    scratch = (
        [pltpu.VMEM((_CH,), jnp.int32)] * _NB +
        [pltpu.VMEM((_CH,), jnp.int32)] * _NB +
        [pltpu.VMEM((_CH,), jnp.float32)] * _NB +
        [pltpu.VMEM((_CH,), jnp.int32)] * _NB +
        [pltpu.VMEM((_CH, d), jnp.float32)] * _NB +
        [pltpu.VMEM((8, d), jnp.float32)] +
        [pltpu.VMEM_SHARED((npad, d), jnp.float32)] +
        [pltpu.SemaphoreType.DMA] * (3 * _NB)
    )

    @functools.partial(
        pl.kernel, mesh=mesh,
        out_type=jax.ShapeDtypeStruct((_NC, npad, d), jnp.float32),
        scratch_types=scratch,
    )
    def body(gidx_h, dst_h, norm_h, tab_h, out_h, *scr):
        cid = lax.axis_index("c")
        sid = lax.axis_index("s")
        wid = sid * _NC + cid
        wbase = wid * epw
        gis = scr[0:_NB]
        dss = scr[_NB:2 * _NB]
        nxs = scr[2 * _NB:3 * _NB]
        dcs = scr[3 * _NB:4 * _NB]
        rows = scr[4 * _NB:5 * _NB]
        zero_v = scr[5 * _NB]
        agg_s = scr[5 * _NB + 1]
        semm = scr[5 * _NB + 2:5 * _NB + 2 + _NB]
        semr = scr[5 * _NB + 2 + _NB:5 * _NB + 2 + 2 * _NB]
        sems = scr[5 * _NB + 2 + 2 * _NB:5 * _NB + 2 + 3 * _NB]

        zvec = jnp.zeros((16,), jnp.float32)
        for i in range(8):
            for j in range(d // 16):
                zero_v[i, pl.ds(j * 16, 16)] = zvec

        def zloop(i, c):
            pltpu.sync_copy(zero_v, agg_s.at[pl.ds(sid * rpt + i * 8, 8)])
            return c
        lax.fori_loop(0, rpt // 8, zloop, 0)
        plsc.subcore_barrier()

        def meta_issue(ci, b):
            base = pl.ds(wbase + ci * _CH, _CH)
            pltpu.async_copy(gidx_h.at[base], gis[b], semm[b])
            pltpu.async_copy(dst_h.at[base], dss[b], semm[b])
            pltpu.async_copy(norm_h.at[base], nxs[b], semm[b])

        def meta_wait(ci, b):
            base = pl.ds(wbase + ci * _CH, _CH)
            pltpu.make_async_copy(gidx_h.at[base], gis[b], semm[b]).wait()
            pltpu.make_async_copy(dst_h.at[base], dss[b], semm[b]).wait()
            pltpu.make_async_copy(norm_h.at[base], nxs[b], semm[b]).wait()

        def rows_issue(b):
            pltpu.async_copy(tab_h.at[gis[b]], rows[b], semr[b])

        def rows_wait(b):
            pltpu.make_async_copy(tab_h.at[gis[b]], rows[b], semr[b]).wait()

        def scat_issue(b):
            for k in range(_CH // 16):
                sl = pl.ds(k * 16, 16)
                dcs[b][sl] = dss[b][sl]
            pltpu.async_copy(rows[b], agg_s.at[dcs[b]], sems[b], add=True)

        def scat_wait(b):
            pltpu.make_async_copy(rows[b], agg_s.at[dcs[b]], sems[b]).wait()

        dnums = lax.GatherDimensionNumbers(
            offset_dims=(), collapsed_slice_dims=(0,),
            start_index_map=(0,))

        def step(cj, b):
            nxt = (b + 1) % _NB

            @pl.when(cj >= _NB - 1)
            def _():
                scat_wait(nxt)

            @pl.when(cj + 1 < nch)
            def _():
                meta_wait(cj + 1, nxt)
                rows_issue(nxt)
            rows_wait(b)

            def scale(g, c2):
                nvec = nxs[b][pl.ds(g * 16, 16)]
                for l in range(16):
                    nv = lax.gather(
                        nvec, jnp.full((16, 1), l, jnp.int32), dnums,
                        slice_sizes=(1,),
                        mode=lax.GatherScatterMode.PROMISE_IN_BOUNDS)
                    e = g * 16 + l
                    for j in range(d // 16):
                        sl = pl.ds(j * 16, 16)
                        rows[b][e, sl] = rows[b][e, sl] * nv
                return c2
            lax.fori_loop(0, _CH // 16, scale, 0)
            scat_issue(b)

            @pl.when(cj + _NB < nch)
            def _():
                meta_issue(cj + _NB, b)

        for k in range(_NB):
            meta_issue(k, k)
        meta_wait(0, 0)
        rows_issue(0)

        def group(i, c):
            ci = i * _NB
            for k in range(_NB):
                step(ci + k, k)
            return c
        lax.fori_loop(0, nch // _NB, group, 0)
        for k in range(1, _NB):
            scat_wait(k)
        plsc.subcore_barrier()

        pltpu.sync_copy(agg_s.at[pl.ds(sid * rpt, rpt)],
                        out_h.at[cid, pl.ds(sid * rpt, rpt)])

    return body(gidx, dst, norm, table_flat)


def kernel(x, edge_index, rel_type, edge_norm, W0, b0, loop0, gate0,
           W1, b1, loop1, gate1):
    n, d = x.shape
    r = W0.shape[0]
    e = rel_type.shape[0]
    npad = -(-n // _BN) * _BN
    eunit = _NW * _CH * _NB
    epad = -(-e // eunit) * eunit

    src = edge_index[0]
    dst = edge_index[1]
    rel = rel_type
    norm = edge_norm[:, 0]
    if epad != e:
        pad = epad - e
        src = jnp.pad(src, (0, pad))
        dst = jnp.pad(dst, (0, pad))
        rel = jnp.pad(rel, (0, pad))
        norm = jnp.pad(norm, (0, pad))
    hpad = jnp.pad(x, ((0, npad - n), (0, 0)))
    gidx = _edge_prep(src, rel, epad, npad).reshape(epad)

    t0 = _make_table(hpad, W0, gate0.reshape(r, 1, d), npad, d, r)
    p0 = _edge_agg(gidx, dst, norm, t0, npad, d)
    h1 = _combine(p0, hpad, loop0, b0.reshape(1, d), True, npad, d)

    t1 = _make_table(h1, W1, gate1.reshape(r, 1, d), npad, d, r)
    p1 = _edge_agg(gidx, dst, norm, t1, npad, d)
    out = _combine(p1, h1, loop1, b1.reshape(1, d), False, npad, d)
    return out[:n]
